# Initial kernel scaffold; baseline (speedup 1.0000x reference)
import jax
import jax.numpy as jnp
from jax.experimental import pallas as pl


def kernel(x, edge_index, W1, b1, W2, b2, W3, b3):
    raise NotImplementedError("write your pallas kernel here")


# speedup vs baseline: 10.0314x; 10.0314x over previous
import functools

import jax
import jax.numpy as jnp
from jax import lax
from jax.experimental import pallas as pl
from jax.experimental.pallas import tpu as pltpu
from jax.experimental.pallas import tpu_sc as plsc

N = 10000
D_IN = 128
D_H = 128
C = 40
C_PAD = 64
E = 320000

NC = 2
NS = 16
NW = NC * NS
CH = 128
CHUNKS_PER_WORKER = 79
EPW = CH * CHUNKS_PER_WORKER
E_PAD = EPW * NW
N_ACC = 10240
RPS = N_ACC // NS

_MESH = plsc.VectorSubcoreMesh(core_axis_name="c", subcore_axis_name="s")


def _deg_body(col_hbm, out_hbm, acc, colv, ones_v, zero_v):
    cid = lax.axis_index("c")
    sid = lax.axis_index("s")
    z16 = jnp.zeros((16,), jnp.float32)
    o16 = jnp.ones((16,), jnp.float32)
    for k in range(CH // 16):
        zero_v[pl.ds(k * 16, 16)] = z16
        ones_v[pl.ds(k * 16, 16)] = o16
    base = sid * RPS

    def zloop(j, carry):
        pltpu.sync_copy(zero_v, acc.at[pl.ds(base + j * CH, CH)])
        return carry

    lax.fori_loop(0, RPS // CH, zloop, 0)
    plsc.subcore_barrier()

    ebase = (cid * NS + sid) * EPW

    def body(j, carry):
        pltpu.sync_copy(col_hbm.at[pl.ds(ebase + j * CH, CH)], colv.at[0])
        pltpu.sync_copy(ones_v, acc.at[colv.at[0]], add=True)
        return carry

    lax.fori_loop(0, CHUNKS_PER_WORKER, body, 0)
    plsc.subcore_barrier()
    pltpu.sync_copy(acc.at[pl.ds(base, RPS)], out_hbm.at[cid, pl.ds(base, RPS)])


_deg = pl.kernel(
    _deg_body,
    out_type=jax.ShapeDtypeStruct((NC, N_ACC), jnp.float32),
    mesh=_MESH,
    scratch_types=[
        pltpu.VMEM_SHARED((N_ACC,), jnp.float32),
        pltpu.VMEM((1, CH), jnp.int32),
        pltpu.VMEM((CH,), jnp.float32),
        pltpu.VMEM((CH,), jnp.float32),
    ],
)


def _scatter_body(width, g_hbm, row_hbm, col_hbm, out_hbm,
                  acc, rowv, colv, valv, zerov, sem):
    cid = lax.axis_index("c")
    sid = lax.axis_index("s")
    z16 = jnp.zeros((16,), jnp.float32)
    for r in range(16):
        for k in range(width // 16):
            zerov[r, pl.ds(k * 16, 16)] = z16
    base = sid * RPS

    def zloop(j, carry):
        pltpu.sync_copy(zerov, acc.at[pl.ds(base + j * 16, 16)])
        return carry

    lax.fori_loop(0, RPS // 16, zloop, 0)
    plsc.subcore_barrier()

    ebase = (cid * NS + sid) * EPW

    def body(j, carry):
        off = ebase + j * CH
        pltpu.sync_copy(row_hbm.at[pl.ds(off, CH)], rowv.at[0])
        pltpu.sync_copy(col_hbm.at[pl.ds(off, CH)], colv.at[0])
        pltpu.async_copy(g_hbm.at[rowv.at[0]], valv.at[0], sem).wait()
        pltpu.sync_copy(valv.at[0], acc.at[colv.at[0]], add=True)
        return carry

    lax.fori_loop(0, CHUNKS_PER_WORKER, body, 0)
    plsc.subcore_barrier()
    pltpu.sync_copy(acc.at[pl.ds(base, RPS)], out_hbm.at[cid, pl.ds(base, RPS)])


def _make_scatter(width):
    extra = {}
    if width < 128:
        extra["compiler_params"] = pltpu.CompilerParams(use_tc_tiling_on_sc=False)
    return pl.kernel(
        functools.partial(_scatter_body, width),
        out_type=jax.ShapeDtypeStruct((NC, N_ACC, width), jnp.float32),
        mesh=_MESH,
        **extra,
        scratch_types=[
            pltpu.VMEM_SHARED((N_ACC, width), jnp.float32),
            pltpu.VMEM((1, CH), jnp.int32),
            pltpu.VMEM((1, CH), jnp.int32),
            pltpu.VMEM((1, CH, width), jnp.float32),
            pltpu.VMEM((16, width), jnp.float32),
            pltpu.SemaphoreType.DMA,
        ],
    )


_scatter_h = _make_scatter(D_H)
_scatter_c = _make_scatter(C_PAD)

R = 1024
NB = (N + R - 1) // R


def _dinv(degp_ref):
    return lax.rsqrt(degp_ref[0] + degp_ref[1] + 1.0)


def _prep_body(degp_ref, x_ref, w_ref, g_ref):
    h = jnp.dot(x_ref[...], w_ref[...], preferred_element_type=jnp.float32)
    g_ref[...] = h * _dinv(degp_ref)


def _combine_mm_body(degp_ref, p_ref, g_ref, b_ref, w_ref, out_ref):
    dinv = _dinv(degp_ref)
    s = p_ref[0] + p_ref[1] + g_ref[...]
    h = jnp.maximum(s * dinv + b_ref[...][None, :], 0.0)
    out_ref[...] = jnp.dot(h, w_ref[...], preferred_element_type=jnp.float32) * dinv


def _combine2_body(degp_ref, p_ref, g_ref, b_ref, w_ref, h2_ref, g3_ref):
    dinv = _dinv(degp_ref)
    s = p_ref[0] + p_ref[1] + g_ref[...]
    h = jnp.maximum(s * dinv + b_ref[...][None, :], 0.0)
    h2_ref[...] = h
    g3_ref[...] = jnp.dot(h, w_ref[...], preferred_element_type=jnp.float32) * dinv


def _combine3_body(degp_ref, p_ref, g_ref, b_ref, out_ref):
    s = p_ref[0] + p_ref[1] + g_ref[...]
    out_ref[...] = s * _dinv(degp_ref) + b_ref[...][None, :]


def _degp_spec():
    return pl.BlockSpec((2, R, 1), lambda i: (0, i, 0))


def _prep(degp3, x, W1):
    return pl.pallas_call(
        _prep_body,
        grid=(NB,),
        in_specs=[_degp_spec(),
                  pl.BlockSpec((R, D_IN), lambda i: (i, 0)),
                  pl.BlockSpec((D_IN, D_H), lambda i: (0, 0))],
        out_specs=pl.BlockSpec((R, D_H), lambda i: (i, 0)),
        out_shape=jax.ShapeDtypeStruct((N, D_H), jnp.float32),
    )(degp3, x, W1)


def _combine_mm(degp3, p, g, b, w):
    return pl.pallas_call(
        _combine_mm_body,
        grid=(NB,),
        in_specs=[_degp_spec(),
                  pl.BlockSpec((2, R, D_H), lambda i: (0, i, 0)),
                  pl.BlockSpec((R, D_H), lambda i: (i, 0)),
                  pl.BlockSpec((D_H,), lambda i: (0,)),
                  pl.BlockSpec((D_H, D_H), lambda i: (0, 0))],
        out_specs=pl.BlockSpec((R, D_H), lambda i: (i, 0)),
        out_shape=jax.ShapeDtypeStruct((N, D_H), jnp.float32),
    )(degp3, p, g, b, w)


def _combine2(degp3, p, g, b, w3p):
    return pl.pallas_call(
        _combine2_body,
        grid=(NB,),
        in_specs=[_degp_spec(),
                  pl.BlockSpec((2, R, D_H), lambda i: (0, i, 0)),
                  pl.BlockSpec((R, D_H), lambda i: (i, 0)),
                  pl.BlockSpec((D_H,), lambda i: (0,)),
                  pl.BlockSpec((D_H, C_PAD), lambda i: (0, 0))],
        out_specs=[pl.BlockSpec((R, D_H), lambda i: (i, 0)),
                   pl.BlockSpec((R, C_PAD), lambda i: (i, 0))],
        out_shape=[jax.ShapeDtypeStruct((N, D_H), jnp.float32),
                   jax.ShapeDtypeStruct((N, C_PAD), jnp.float32)],
    )(degp3, p, g, b, w3p)


def _combine3(degp3, p, g, b):
    return pl.pallas_call(
        _combine3_body,
        grid=(NB,),
        in_specs=[_degp_spec(),
                  pl.BlockSpec((2, R, C_PAD), lambda i: (0, i, 0)),
                  pl.BlockSpec((R, C_PAD), lambda i: (i, 0)),
                  pl.BlockSpec((C_PAD,), lambda i: (0,))],
        out_specs=pl.BlockSpec((R, C_PAD), lambda i: (i, 0)),
        out_shape=jax.ShapeDtypeStruct((N, C_PAD), jnp.float32),
    )(degp3, p, g, b)


def kernel(x, edge_index, W1, b1, W2, b2, W3, b3):
    row_p = jnp.concatenate([edge_index[0], jnp.zeros((E_PAD - E,), jnp.int32)])
    col_p = jnp.concatenate([edge_index[1], jnp.full((E_PAD - E,), N, jnp.int32)])

    degp = _deg(col_p)
    degp3 = degp.reshape(NC, N_ACC, 1)

    g1 = _prep(degp3, x, W1)
    p1 = _scatter_h(g1, row_p, col_p)
    g2 = _combine_mm(degp3, p1, g1, b1, W2)
    p2 = _scatter_h(g2, row_p, col_p)
    w3p = jnp.pad(W3, ((0, 0), (0, C_PAD - C)))
    h2, g3 = _combine2(degp3, p2, g2, b2, w3p)
    p3 = _scatter_c(g3, row_p, col_p)
    b3p = jnp.pad(b3, (0, C_PAD - C))
    out = _combine3(degp3, p3, g3, b3p)
    return (h2, out[:, :C])
